# baseline (speedup 1.0000x reference)
import jax
import jax.numpy as jnp
from jax.experimental import pallas as pl
from jax.experimental.pallas import tpu as pltpu

N = 10000
VAR = 4
NIN = 256
NZ = 100
NC = 10

_BM_AE = 2000
_BM_G = 400
_NBG = N // _BM_G


def _mm(a, b):
    return jax.lax.dot_general(a, b, (((1,), (0,)), ((), ())),
                               preferred_element_type=jnp.float32)


def _ae_body(x_ref, eps_ref, t0_ref, c0b_ref, t1_ref, c1b_ref,
             f1w_ref, f1b_ref, f2w_ref, f2b_ref, f31w_ref, f31b_ref,
             f21w_ref, f21b_ref, f22w_ref, f22b_ref,
             f3w_ref, f3b_ref, f32w_ref, f32b_ref, f4w_ref, f4b_ref,
             g1w_ref,
             out0_ref, mu_ref, logvar_ref, s1_ref):
    x = x_ref[...]
    pc = jax.lax.dot_general(x, t0_ref[...], (((2,), (1,)), ((1,), (0,))),
                             preferred_element_type=jnp.float32)
    pro = jnp.sum(pc, axis=0) + c0b_ref[0:1, 0:1]
    h1 = jax.nn.relu(_mm(pro, f1w_ref[...]) + f1b_ref[...])
    h2 = jax.nn.relu(_mm(h1, f2w_ref[...]) + f2b_ref[...])
    h3 = jax.nn.relu(_mm(h2, f31w_ref[...]) + f31b_ref[...])
    mu = _mm(h3, f21w_ref[...]) + f21b_ref[...]
    logvar = _mm(h3, f22w_ref[...]) + f22b_ref[...]
    std = jnp.exp(0.5 * logvar)
    z = eps_ref[...] * std + mu
    d3 = jax.nn.relu(_mm(z, f3w_ref[...]) + f3b_ref[...])
    d4 = jax.nn.relu(_mm(d3, f32w_ref[...]) + f32b_ref[...])
    recon = jax.nn.sigmoid(_mm(d4, f4w_ref[...]) + f4b_ref[...])
    for co in range(VAR):
        out0_ref[:, co, :] = _mm(recon, t1_ref[co]) + c1b_ref[0:1, co:co + 1]
    mu_ref[...] = mu
    logvar_ref[...] = logvar
    s1_ref[...] = _mm(pro, g1w_ref[...])


def _gnn_body(adj_ref, s1_ref, g3w_ref, g4w_ref, g5w_ref,
              fccw_ref, fccb_ref, pred_ref, sa_ref, sb_ref):
    l = pl.program_id(0)
    i = pl.program_id(1)
    rows = pl.ds(i * _BM_G, _BM_G)

    def adj_mm(s_ref):
        return _mm(adj_ref[...], s_ref[...])

    @pl.when(l == 0)
    def _():
        sa_ref[rows, :] = s1_ref[...]

    @pl.when(l == 1)
    def _():
        h = jax.nn.relu(adj_mm(sa_ref))
        sb_ref[rows, :] = _mm(h, g3w_ref[...])

    @pl.when(l == 2)
    def _():
        h = jax.nn.relu(adj_mm(sb_ref))
        sa_ref[rows, :] = _mm(h, g4w_ref[...])

    @pl.when(l == 3)
    def _():
        h = adj_mm(sa_ref)
        sb_ref[rows, :] = _mm(h, g5w_ref[...])

    @pl.when(l == 4)
    def _():
        h = adj_mm(sb_ref)
        logits = _mm(h, fccw_ref[...]) + fccb_ref[...]
        m = jnp.max(logits, axis=1, keepdims=True)
        e = jnp.exp(logits - m)
        pred_ref[...] = e / jnp.sum(e, axis=1, keepdims=True)


def _full_spec(shape):
    nd = len(shape)
    return pl.BlockSpec(shape, lambda l, i, _n=nd: (0,) * _n)


def kernel(x, adj, eps, conv0_w, conv0_b, fc1_w, fc1_b, fc2_w, fc2_b,
           fc31_w, fc31_b, fc21_w, fc21_b, fc22_w, fc22_b, fc3_w, fc3_b,
           fc32_w, fc32_b, fc4_w, fc4_b, conv1_w, conv1_b,
           g1_w, g3_w, g4_w, g5_w, fcc_w, fcc_b):
    f32 = jnp.float32
    c0w = conv0_w.reshape(VAR, 3)
    c0b = conv0_b.reshape(1, 1)
    c1w = conv1_w.reshape(VAR, 3)
    c1b = conv1_b.reshape(1, VAR)
    e_up = jnp.eye(NIN, k=1, dtype=f32)
    e_d = jnp.eye(NIN, dtype=f32)
    e_dn = jnp.eye(NIN, k=-1, dtype=f32)
    t0 = (c0w[:, 0, None, None] * e_up + c0w[:, 1, None, None] * e_d
          + c0w[:, 2, None, None] * e_dn)
    t1 = (c1w[:, 0, None, None] * e_up + c1w[:, 1, None, None] * e_d
          + c1w[:, 2, None, None] * e_dn)

    nb = N // _BM_AE
    ae_inputs = (x, eps, t0, c0b, t1, c1b,
                 fc1_w, fc1_b.reshape(1, -1), fc2_w, fc2_b.reshape(1, -1),
                 fc31_w, fc31_b.reshape(1, -1), fc21_w, fc21_b.reshape(1, -1),
                 fc22_w, fc22_b.reshape(1, -1), fc3_w, fc3_b.reshape(1, -1),
                 fc32_w, fc32_b.reshape(1, -1), fc4_w, fc4_b.reshape(1, -1),
                 g1_w)
    ae_in_specs = [
        pl.BlockSpec((_BM_AE, VAR, NIN), lambda i: (i, 0, 0)),
        pl.BlockSpec((_BM_AE, NZ), lambda i: (i, 0)),
    ] + [pl.BlockSpec(a.shape, lambda i, _n=a.ndim: (0,) * _n)
         for a in ae_inputs[2:]]
    out0, mu, logvar, s1 = pl.pallas_call(
        _ae_body,
        grid=(nb,),
        in_specs=ae_in_specs,
        out_specs=[
            pl.BlockSpec((_BM_AE, VAR, NIN), lambda i: (i, 0, 0)),
            pl.BlockSpec((_BM_AE, NZ), lambda i: (i, 0)),
            pl.BlockSpec((_BM_AE, NZ), lambda i: (i, 0)),
            pl.BlockSpec((_BM_AE, NZ), lambda i: (i, 0)),
        ],
        out_shape=[
            jax.ShapeDtypeStruct((N, VAR, NIN), f32),
            jax.ShapeDtypeStruct((N, NZ), f32),
            jax.ShapeDtypeStruct((N, NZ), f32),
            jax.ShapeDtypeStruct((N, NZ), f32),
        ],
    )(*ae_inputs)

    gnn_inputs = (adj, s1, g3_w, g4_w, g5_w, fcc_w, fcc_b.reshape(1, NC))
    predict = pl.pallas_call(
        _gnn_body,
        grid=(5, _NBG),
        in_specs=[
            pl.BlockSpec((_BM_G, N),
                         lambda l, i: (jax.lax.select(l == 0, 0, i), 0)),
            pl.BlockSpec((_BM_G, NZ),
                         lambda l, i: (jax.lax.select(l == 0, i, 0), 0)),
        ] + [_full_spec(a.shape) for a in gnn_inputs[2:]],
        out_specs=pl.BlockSpec((_BM_G, NC), lambda l, i: (i, 0)),
        out_shape=jax.ShapeDtypeStruct((N, NC), f32),
        scratch_shapes=[
            pltpu.VMEM((N, NZ), f32),
            pltpu.VMEM((N, NZ), f32),
        ],
    )(*gnn_inputs)

    return (out0, predict, mu, logvar)


# speedup vs baseline: 1.0031x; 1.0031x over previous
import jax
import jax.numpy as jnp
from jax.experimental import pallas as pl
from jax.experimental.pallas import tpu as pltpu

N = 10000
VAR = 4
NIN = 256
NZ = 100
NC = 10

_BM_AE = 1000
_BM_G = 400
_NBG = N // _BM_G


def _mm(a, b):
    return jax.lax.dot_general(a, b, (((1,), (0,)), ((), ())),
                               preferred_element_type=jnp.float32)


def _ae_body(x_ref, eps_ref, t0_ref, c0b_ref, t1_ref, c1b_ref,
             f1w_ref, f1b_ref, f2w_ref, f2b_ref, f31w_ref, f31b_ref,
             f21w_ref, f21b_ref, f22w_ref, f22b_ref,
             f3w_ref, f3b_ref, f32w_ref, f32b_ref, f4w_ref, f4b_ref,
             g1w_ref,
             out0_ref, mu_ref, logvar_ref, s1_ref):
    x = x_ref[...]
    pc = jax.lax.dot_general(x, t0_ref[...], (((2,), (1,)), ((1,), (0,))),
                             preferred_element_type=jnp.float32)
    pro = jnp.sum(pc, axis=0) + c0b_ref[0:1, 0:1]
    h1 = jax.nn.relu(_mm(pro, f1w_ref[...]) + f1b_ref[...])
    h2 = jax.nn.relu(_mm(h1, f2w_ref[...]) + f2b_ref[...])
    h3 = jax.nn.relu(_mm(h2, f31w_ref[...]) + f31b_ref[...])
    mu = _mm(h3, f21w_ref[...]) + f21b_ref[...]
    logvar = _mm(h3, f22w_ref[...]) + f22b_ref[...]
    std = jnp.exp(0.5 * logvar)
    z = eps_ref[...] * std + mu
    d3 = jax.nn.relu(_mm(z, f3w_ref[...]) + f3b_ref[...])
    d4 = jax.nn.relu(_mm(d3, f32w_ref[...]) + f32b_ref[...])
    recon = jax.nn.sigmoid(_mm(d4, f4w_ref[...]) + f4b_ref[...])
    for co in range(VAR):
        out0_ref[:, co, :] = _mm(recon, t1_ref[co]) + c1b_ref[0:1, co:co + 1]
    mu_ref[...] = mu
    logvar_ref[...] = logvar
    s1_ref[...] = _mm(pro, g1w_ref[...])


def _gnn_body(adj_ref, s1_ref, g3w_ref, g4w_ref, g5w_ref,
              fccw_ref, fccb_ref, pred_ref, sa_ref, sb_ref):
    l = pl.program_id(0)
    i = pl.program_id(1)
    rows = pl.ds(i * _BM_G, _BM_G)

    def adj_mm(s_ref):
        return _mm(adj_ref[...], s_ref[...])

    @pl.when(l == 0)
    def _():
        sa_ref[rows, :] = s1_ref[...]

    @pl.when(l == 1)
    def _():
        h = jax.nn.relu(adj_mm(sa_ref))
        sb_ref[rows, :] = _mm(h, g3w_ref[...])

    @pl.when(l == 2)
    def _():
        h = jax.nn.relu(adj_mm(sb_ref))
        sa_ref[rows, :] = _mm(h, g4w_ref[...])

    @pl.when(l == 3)
    def _():
        h = adj_mm(sa_ref)
        sb_ref[rows, :] = _mm(h, g5w_ref[...])

    @pl.when(l == 4)
    def _():
        h = adj_mm(sb_ref)
        logits = _mm(h, fccw_ref[...]) + fccb_ref[...]
        m = jnp.max(logits, axis=1, keepdims=True)
        e = jnp.exp(logits - m)
        pred_ref[...] = e / jnp.sum(e, axis=1, keepdims=True)


def _full_spec(shape):
    nd = len(shape)
    return pl.BlockSpec(shape, lambda l, i, _n=nd: (0,) * _n)


def kernel(x, adj, eps, conv0_w, conv0_b, fc1_w, fc1_b, fc2_w, fc2_b,
           fc31_w, fc31_b, fc21_w, fc21_b, fc22_w, fc22_b, fc3_w, fc3_b,
           fc32_w, fc32_b, fc4_w, fc4_b, conv1_w, conv1_b,
           g1_w, g3_w, g4_w, g5_w, fcc_w, fcc_b):
    f32 = jnp.float32
    c0w = conv0_w.reshape(VAR, 3)
    c0b = conv0_b.reshape(1, 1)
    c1w = conv1_w.reshape(VAR, 3)
    c1b = conv1_b.reshape(1, VAR)
    e_up = jnp.eye(NIN, k=1, dtype=f32)
    e_d = jnp.eye(NIN, dtype=f32)
    e_dn = jnp.eye(NIN, k=-1, dtype=f32)
    t0 = (c0w[:, 0, None, None] * e_up + c0w[:, 1, None, None] * e_d
          + c0w[:, 2, None, None] * e_dn)
    t1 = (c1w[:, 0, None, None] * e_up + c1w[:, 1, None, None] * e_d
          + c1w[:, 2, None, None] * e_dn)

    nb = N // _BM_AE
    ae_inputs = (x, eps, t0, c0b, t1, c1b,
                 fc1_w, fc1_b.reshape(1, -1), fc2_w, fc2_b.reshape(1, -1),
                 fc31_w, fc31_b.reshape(1, -1), fc21_w, fc21_b.reshape(1, -1),
                 fc22_w, fc22_b.reshape(1, -1), fc3_w, fc3_b.reshape(1, -1),
                 fc32_w, fc32_b.reshape(1, -1), fc4_w, fc4_b.reshape(1, -1),
                 g1_w)
    ae_in_specs = [
        pl.BlockSpec((_BM_AE, VAR, NIN), lambda i: (i, 0, 0)),
        pl.BlockSpec((_BM_AE, NZ), lambda i: (i, 0)),
    ] + [pl.BlockSpec(a.shape, lambda i, _n=a.ndim: (0,) * _n)
         for a in ae_inputs[2:]]
    out0, mu, logvar, s1 = pl.pallas_call(
        _ae_body,
        grid=(nb,),
        in_specs=ae_in_specs,
        out_specs=[
            pl.BlockSpec((_BM_AE, VAR, NIN), lambda i: (i, 0, 0)),
            pl.BlockSpec((_BM_AE, NZ), lambda i: (i, 0)),
            pl.BlockSpec((_BM_AE, NZ), lambda i: (i, 0)),
            pl.BlockSpec((_BM_AE, NZ), lambda i: (i, 0)),
        ],
        out_shape=[
            jax.ShapeDtypeStruct((N, VAR, NIN), f32),
            jax.ShapeDtypeStruct((N, NZ), f32),
            jax.ShapeDtypeStruct((N, NZ), f32),
            jax.ShapeDtypeStruct((N, NZ), f32),
        ],
    )(*ae_inputs)

    gnn_inputs = (adj, s1, g3_w, g4_w, g5_w, fcc_w, fcc_b.reshape(1, NC))
    predict = pl.pallas_call(
        _gnn_body,
        grid=(5, _NBG),
        in_specs=[
            pl.BlockSpec((_BM_G, N),
                         lambda l, i: (jax.lax.select(l == 0, 0, i), 0)),
            pl.BlockSpec((_BM_G, NZ),
                         lambda l, i: (jax.lax.select(l == 0, i, 0), 0)),
        ] + [_full_spec(a.shape) for a in gnn_inputs[2:]],
        out_specs=pl.BlockSpec((_BM_G, NC), lambda l, i: (i, 0)),
        out_shape=jax.ShapeDtypeStruct((N, NC), f32),
        scratch_shapes=[
            pltpu.VMEM((N, NZ), f32),
            pltpu.VMEM((N, NZ), f32),
        ],
    )(*gnn_inputs)

    return (out0, predict, mu, logvar)


# speedup vs baseline: 1.0112x; 1.0081x over previous
import jax
import jax.numpy as jnp
from jax.experimental import pallas as pl
from jax.experimental.pallas import tpu as pltpu

N = 10000
VAR = 4
NIN = 256
NZ = 100
NC = 10

_BM_AE = 1000
_BM_G = 400
_NBG = N // _BM_G


def _mm(a, b):
    return jax.lax.dot_general(a, b, (((1,), (0,)), ((), ())),
                               preferred_element_type=jnp.float32)


def _ae_body(x_ref, eps_ref, t0_ref, c0b_ref, t1_ref, c1b_ref,
             f1w_ref, f1b_ref, f2w_ref, f2b_ref, f31w_ref, f31b_ref,
             f21w_ref, f21b_ref, f22w_ref, f22b_ref,
             f3w_ref, f3b_ref, f32w_ref, f32b_ref, f4w_ref, f4b_ref,
             g1w_ref,
             out0_ref, mu_ref, logvar_ref, s1_ref):
    x = x_ref[...]
    pc = jax.lax.dot_general(x, t0_ref[...], (((2,), (1,)), ((1,), (0,))),
                             preferred_element_type=jnp.float32)
    pro = jnp.sum(pc, axis=0) + c0b_ref[0:1, 0:1]
    h1 = jax.nn.relu(_mm(pro, f1w_ref[...]) + f1b_ref[...])
    h2 = jax.nn.relu(_mm(h1, f2w_ref[...]) + f2b_ref[...])
    h3 = jax.nn.relu(_mm(h2, f31w_ref[...]) + f31b_ref[...])
    mu = _mm(h3, f21w_ref[...]) + f21b_ref[...]
    logvar = _mm(h3, f22w_ref[...]) + f22b_ref[...]
    std = jnp.exp(0.5 * logvar)
    z = eps_ref[...] * std + mu
    d3 = jax.nn.relu(_mm(z, f3w_ref[...]) + f3b_ref[...])
    d4 = jax.nn.relu(_mm(d3, f32w_ref[...]) + f32b_ref[...])
    recon = jax.nn.sigmoid(_mm(d4, f4w_ref[...]) + f4b_ref[...])
    for co in range(VAR):
        out0_ref[:, co, :] = _mm(recon, t1_ref[co]) + c1b_ref[0:1, co:co + 1]
    mu_ref[...] = mu
    logvar_ref[...] = logvar
    s1_ref[...] = _mm(pro, g1w_ref[...])


_NCOPY = 10


def _gnn_body(adj_ref, s1_ref, g3w_ref, g4w_ref, g5w_ref,
              fccw_ref, fccb_ref, pred_ref, sa_ref, sb_ref):
    t = pl.program_id(0)
    i = _row_idx(t)
    rows = pl.ds(i * _BM_G, _BM_G)

    def adj_mm(s_ref):
        return _mm(adj_ref[...], s_ref[...])

    @pl.when(t < _NCOPY)
    def _():
        sa_ref[pl.ds(t * (N // _NCOPY), N // _NCOPY), :] = s1_ref[...]

    @pl.when((t >= _NCOPY) & (t < _NCOPY + _NBG))
    def _():
        h = jax.nn.relu(adj_mm(sa_ref))
        sb_ref[rows, :] = _mm(h, g3w_ref[...])

    @pl.when((t >= _NCOPY + _NBG) & (t < _NCOPY + 2 * _NBG))
    def _():
        h = jax.nn.relu(adj_mm(sb_ref))
        sa_ref[rows, :] = _mm(h, g4w_ref[...])

    @pl.when((t >= _NCOPY + 2 * _NBG) & (t < _NCOPY + 3 * _NBG))
    def _():
        h = adj_mm(sa_ref)
        sb_ref[rows, :] = _mm(h, g5w_ref[...])

    @pl.when(t >= _NCOPY + 3 * _NBG)
    def _():
        h = adj_mm(sb_ref)
        logits = _mm(h, fccw_ref[...]) + fccb_ref[...]
        m = jnp.max(logits, axis=1, keepdims=True)
        e = jnp.exp(logits - m)
        pred_ref[...] = e / jnp.sum(e, axis=1, keepdims=True)


def _full_spec(shape):
    nd = len(shape)
    return pl.BlockSpec(shape, lambda t, _n=nd: (0,) * _n)


def _row_idx(t):
    return jax.lax.rem(jax.lax.max(t - _NCOPY, 0), _NBG)


def kernel(x, adj, eps, conv0_w, conv0_b, fc1_w, fc1_b, fc2_w, fc2_b,
           fc31_w, fc31_b, fc21_w, fc21_b, fc22_w, fc22_b, fc3_w, fc3_b,
           fc32_w, fc32_b, fc4_w, fc4_b, conv1_w, conv1_b,
           g1_w, g3_w, g4_w, g5_w, fcc_w, fcc_b):
    f32 = jnp.float32
    c0w = conv0_w.reshape(VAR, 3)
    c0b = conv0_b.reshape(1, 1)
    c1w = conv1_w.reshape(VAR, 3)
    c1b = conv1_b.reshape(1, VAR)
    e_up = jnp.eye(NIN, k=1, dtype=f32)
    e_d = jnp.eye(NIN, dtype=f32)
    e_dn = jnp.eye(NIN, k=-1, dtype=f32)
    t0 = (c0w[:, 0, None, None] * e_up + c0w[:, 1, None, None] * e_d
          + c0w[:, 2, None, None] * e_dn)
    t1 = (c1w[:, 0, None, None] * e_up + c1w[:, 1, None, None] * e_d
          + c1w[:, 2, None, None] * e_dn)

    nb = N // _BM_AE
    ae_inputs = (x, eps, t0, c0b, t1, c1b,
                 fc1_w, fc1_b.reshape(1, -1), fc2_w, fc2_b.reshape(1, -1),
                 fc31_w, fc31_b.reshape(1, -1), fc21_w, fc21_b.reshape(1, -1),
                 fc22_w, fc22_b.reshape(1, -1), fc3_w, fc3_b.reshape(1, -1),
                 fc32_w, fc32_b.reshape(1, -1), fc4_w, fc4_b.reshape(1, -1),
                 g1_w)
    ae_in_specs = [
        pl.BlockSpec((_BM_AE, VAR, NIN), lambda i: (i, 0, 0)),
        pl.BlockSpec((_BM_AE, NZ), lambda i: (i, 0)),
    ] + [pl.BlockSpec(a.shape, lambda i, _n=a.ndim: (0,) * _n)
         for a in ae_inputs[2:]]
    out0, mu, logvar, s1 = pl.pallas_call(
        _ae_body,
        grid=(nb,),
        in_specs=ae_in_specs,
        out_specs=[
            pl.BlockSpec((_BM_AE, VAR, NIN), lambda i: (i, 0, 0)),
            pl.BlockSpec((_BM_AE, NZ), lambda i: (i, 0)),
            pl.BlockSpec((_BM_AE, NZ), lambda i: (i, 0)),
            pl.BlockSpec((_BM_AE, NZ), lambda i: (i, 0)),
        ],
        out_shape=[
            jax.ShapeDtypeStruct((N, VAR, NIN), f32),
            jax.ShapeDtypeStruct((N, NZ), f32),
            jax.ShapeDtypeStruct((N, NZ), f32),
            jax.ShapeDtypeStruct((N, NZ), f32),
        ],
    )(*ae_inputs)

    gnn_inputs = (adj, s1, g3_w, g4_w, g5_w, fcc_w, fcc_b.reshape(1, NC))
    predict = pl.pallas_call(
        _gnn_body,
        grid=(_NCOPY + 4 * _NBG,),
        in_specs=[
            pl.BlockSpec((_BM_G, N), lambda t: (_row_idx(t), 0)),
            pl.BlockSpec((N // _NCOPY, NZ),
                         lambda t: (jax.lax.min(t, _NCOPY - 1), 0)),
        ] + [_full_spec(a.shape) for a in gnn_inputs[2:]],
        out_specs=pl.BlockSpec((_BM_G, NC), lambda t: (_row_idx(t), 0)),
        out_shape=jax.ShapeDtypeStruct((N, NC), f32),
        scratch_shapes=[
            pltpu.VMEM((N, NZ), f32),
            pltpu.VMEM((N, NZ), f32),
        ],
    )(*gnn_inputs)

    return (out0, predict, mu, logvar)
